# baseline (speedup 1.0000x reference)
from functools import partial

import jax
import jax.numpy as jnp
from jax.experimental import pallas as pl
from jax.experimental.pallas import tpu as pltpu
from jax.sharding import PartitionSpec as P


def _gc_body(adj_ref, x_ref, w_ref, out_ref, y_ref):
    @pl.when(pl.program_id(0) == 0)
    def _():
        y_ref[...] = jax.lax.dot_general(
            x_ref[...], w_ref[...],
            dimension_numbers=(((1,), (0,)), ((), ())),
            precision=jax.lax.Precision.DEFAULT,
            preferred_element_type=jnp.float32)

    out_ref[...] = jax.lax.dot_general(
        adj_ref[...], y_ref[...],
        dimension_numbers=(((1,), (0,)), ((), ())),
        precision=jax.lax.Precision.DEFAULT,
        preferred_element_type=jnp.float32)


def _local_gc(adj, x, w, bm):
    m = adj.shape[0]
    n, d_in = x.shape
    d_out = w.shape[1]
    return pl.pallas_call(
        _gc_body,
        grid=(m // bm,),
        in_specs=[
            pl.BlockSpec((bm, n), lambda i: (i, 0)),
            pl.BlockSpec((n, d_in), lambda i: (0, 0)),
            pl.BlockSpec((d_in, d_out), lambda i: (0, 0)),
        ],
        out_specs=pl.BlockSpec((bm, d_out), lambda i: (i, 0)),
        out_shape=jax.ShapeDtypeStruct((m, d_out), jnp.float32),
        scratch_shapes=[pltpu.VMEM((n, d_out), jnp.float32)],
        compiler_params=pltpu.CompilerParams(
            dimension_semantics=("arbitrary",)),
    )(adj, x, w)


def kernel(input, adj, weight):
    m = adj.shape[0]
    n_dev = jax.device_count()
    if n_dev > 1 and m % (n_dev * 8) == 0:
        mesh = jax.make_mesh((n_dev,), ("d",))
        adj_s = jax.reshard(adj, jax.NamedSharding(mesh, P("d", None)))
        x_s = jax.reshard(input, jax.NamedSharding(mesh, P(None, None)))
        w_s = jax.reshard(weight, jax.NamedSharding(mesh, P(None, None)))
        shard = jax.shard_map(
            partial(_local_gc, bm=200),
            mesh=mesh,
            in_specs=(P("d", None), P(None, None), P(None, None)),
            out_specs=P("d", None),
            check_vma=False,
        )
        return shard(adj_s, x_s, w_s)
    return _local_gc(adj, input, weight, 400)


# speedup vs baseline: 5.1626x; 5.1626x over previous
import jax
import jax.numpy as jnp
from jax.experimental import pallas as pl
from jax.experimental.pallas import tpu as pltpu

_CHUNKS = [48, 352] + [400] * 24
_STARTS = [sum(_CHUNKS[:i]) for i in range(len(_CHUNKS))]
_MAXR = max(_CHUNKS)


def _dot(a, b):
    return jax.lax.dot_general(
        a, b,
        dimension_numbers=(((1,), (0,)), ((), ())),
        precision=jax.lax.Precision.DEFAULT,
        preferred_element_type=jnp.float32)


def _body(adj_hbm, x_hbm, w_ref, out_hbm,
          buf0, buf1, xv, yv, ob0, ob1, in_sems, x_sem, out_sems):
    bufs = (buf0, buf1)
    obs = (ob0, ob1)
    nk = len(_CHUNKS)

    def in_copy(k):
        r, s = _CHUNKS[k], _STARTS[k]
        return pltpu.make_async_copy(
            adj_hbm.at[pl.ds(s, r), :],
            bufs[k % 2].at[pl.ds(0, r), :],
            in_sems.at[k % 2])

    def out_copy(k):
        r, s = _CHUNKS[k], _STARTS[k]
        return pltpu.make_async_copy(
            obs[k % 2].at[pl.ds(0, r), :],
            out_hbm.at[pl.ds(s, r), :],
            out_sems.at[k % 2])

    x_copy = pltpu.make_async_copy(x_hbm, xv, x_sem)
    x_copy.start()
    in_copy(0).start()
    in_copy(1).start()
    x_copy.wait()
    yv[...] = _dot(xv[...], w_ref[...])

    for k in range(nk):
        r = _CHUNKS[k]
        b, o = bufs[k % 2], obs[k % 2]
        in_copy(k).wait()
        if k >= 2:
            out_copy(k - 2).wait()
        o[pl.ds(0, r), :] = _dot(b[pl.ds(0, r), :], yv[...])
        if k + 2 < nk:
            in_copy(k + 2).start()
        out_copy(k).start()

    out_copy(nk - 2).wait()
    out_copy(nk - 1).wait()


def kernel(input, adj, weight):
    n, d_in = input.shape
    m = adj.shape[0]
    d_out = weight.shape[1]

    return pl.pallas_call(
        _body,
        in_specs=[
            pl.BlockSpec(memory_space=pltpu.MemorySpace.HBM),
            pl.BlockSpec(memory_space=pltpu.MemorySpace.HBM),
            pl.BlockSpec(memory_space=pltpu.MemorySpace.VMEM),
        ],
        out_specs=pl.BlockSpec(memory_space=pltpu.MemorySpace.HBM),
        out_shape=jax.ShapeDtypeStruct((m, d_out), jnp.float32),
        scratch_shapes=[
            pltpu.VMEM((_MAXR, n), jnp.float32),
            pltpu.VMEM((_MAXR, n), jnp.float32),
            pltpu.VMEM((n, d_in), jnp.float32),
            pltpu.VMEM((n, d_out), jnp.float32),
            pltpu.VMEM((_MAXR, d_out), jnp.float32),
            pltpu.VMEM((_MAXR, d_out), jnp.float32),
            pltpu.SemaphoreType.DMA((2,)),
            pltpu.SemaphoreType.DMA,
            pltpu.SemaphoreType.DMA((2,)),
        ],
    )(adj, input, weight)


# speedup vs baseline: 5.4114x; 1.0482x over previous
import jax
import jax.numpy as jnp
from jax.experimental import pallas as pl
from jax.experimental.pallas import tpu as pltpu

_CHUNKS = [48, 352] + [400] * 23 + [256, 144]
_STARTS = [sum(_CHUNKS[:i]) for i in range(len(_CHUNKS))]
_MAXR = max(_CHUNKS)


def _dot(a, b):
    return jax.lax.dot_general(
        a, b,
        dimension_numbers=(((1,), (0,)), ((), ())),
        precision=jax.lax.Precision.DEFAULT,
        preferred_element_type=jnp.float32)


def _body(adj_hbm, x_hbm, w_ref, out_hbm,
          buf0, buf1, buf2, xv, yv, ob0, ob1, in_sems, x_sem, out_sems):
    bufs = (buf0, buf1, buf2)
    obs = (ob0, ob1)
    nk = len(_CHUNKS)

    def in_copy(k):
        r, s = _CHUNKS[k], _STARTS[k]
        return pltpu.make_async_copy(
            adj_hbm.at[pl.ds(s, r), :],
            bufs[k % 3].at[pl.ds(0, r), :],
            in_sems.at[k % 3])

    def out_copy(k):
        r, s = _CHUNKS[k], _STARTS[k]
        return pltpu.make_async_copy(
            obs[k % 2].at[pl.ds(0, r), :],
            out_hbm.at[pl.ds(s, r), :],
            out_sems.at[k % 2])

    x_copy = pltpu.make_async_copy(x_hbm, xv, x_sem)
    x_copy.start()
    in_copy(0).start()
    in_copy(1).start()
    x_copy.wait()
    yv[...] = _dot(xv[...], w_ref[...])

    for k in range(nk):
        r = _CHUNKS[k]
        b, o = bufs[k % 3], obs[k % 2]
        in_copy(k).wait()
        if k + 2 < nk:
            in_copy(k + 2).start()
        if k >= 2:
            out_copy(k - 2).wait()
        o[pl.ds(0, r), :] = _dot(b[pl.ds(0, r), :], yv[...])
        out_copy(k).start()

    out_copy(nk - 2).wait()
    out_copy(nk - 1).wait()


def kernel(input, adj, weight):
    n, d_in = input.shape
    m = adj.shape[0]
    d_out = weight.shape[1]

    return pl.pallas_call(
        _body,
        in_specs=[
            pl.BlockSpec(memory_space=pltpu.MemorySpace.HBM),
            pl.BlockSpec(memory_space=pltpu.MemorySpace.HBM),
            pl.BlockSpec(memory_space=pltpu.MemorySpace.VMEM),
        ],
        out_specs=pl.BlockSpec(memory_space=pltpu.MemorySpace.HBM),
        out_shape=jax.ShapeDtypeStruct((m, d_out), jnp.float32),
        scratch_shapes=[
            pltpu.VMEM((_MAXR, n), jnp.float32),
            pltpu.VMEM((_MAXR, n), jnp.float32),
            pltpu.VMEM((_MAXR, n), jnp.float32),
            pltpu.VMEM((n, d_in), jnp.float32),
            pltpu.VMEM((n, d_out), jnp.float32),
            pltpu.VMEM((_MAXR, d_out), jnp.float32),
            pltpu.VMEM((_MAXR, d_out), jnp.float32),
            pltpu.SemaphoreType.DMA((3,)),
            pltpu.SemaphoreType.DMA,
            pltpu.SemaphoreType.DMA((2,)),
        ],
        compiler_params=pltpu.CompilerParams(
            vmem_limit_bytes=64 * 1024 * 1024),
    )(adj, input, weight)


# speedup vs baseline: 5.5085x; 1.0179x over previous
import jax
import jax.numpy as jnp
from jax.experimental import pallas as pl
from jax.experimental.pallas import tpu as pltpu

_BM = 400
_H = _BM // 2


def _dot(a, b):
    return jax.lax.dot_general(
        a, b,
        dimension_numbers=(((1,), (0,)), ((), ())),
        precision=jax.lax.Precision.DEFAULT,
        preferred_element_type=jnp.float32)


def _gc_body(a0_ref, a1_ref, x_ref, w_ref, out_ref, y_ref):
    @pl.when(pl.program_id(0) == 0)
    def _():
        y_ref[...] = _dot(x_ref[...], w_ref[...])

    out_ref[0:_H, :] = _dot(a0_ref[...], y_ref[...])
    out_ref[_H:_BM, :] = _dot(a1_ref[...], y_ref[...])


def kernel(input, adj, weight):
    n, d_in = input.shape
    m = adj.shape[0]
    d_out = weight.shape[1]

    return pl.pallas_call(
        _gc_body,
        grid=(m // _BM,),
        in_specs=[
            pl.BlockSpec((_H, n), lambda i: (2 * i, 0)),
            pl.BlockSpec((_H, n), lambda i: (2 * i + 1, 0)),
            pl.BlockSpec((n, d_in), lambda i: (0, 0)),
            pl.BlockSpec((d_in, d_out), lambda i: (0, 0)),
        ],
        out_specs=pl.BlockSpec((_BM, d_out), lambda i: (i, 0)),
        out_shape=jax.ShapeDtypeStruct((m, d_out), jnp.float32),
        scratch_shapes=[pltpu.VMEM((n, d_out), jnp.float32)],
        compiler_params=pltpu.CompilerParams(
            dimension_semantics=("arbitrary",)),
    )(adj, adj, input, weight)


# speedup vs baseline: 5.5098x; 1.0003x over previous
import jax
import jax.numpy as jnp
from jax.experimental import pallas as pl
from jax.experimental.pallas import tpu as pltpu

_BM = 400


def _gc_body(adj_ref, x_ref, w_ref, out_ref, y_ref):
    @pl.when(pl.program_id(0) == 0)
    def _():
        y_ref[...] = jax.lax.dot_general(
            x_ref[...], w_ref[...],
            dimension_numbers=(((1,), (0,)), ((), ())),
            precision=jax.lax.Precision.DEFAULT,
            preferred_element_type=jnp.float32)

    out_ref[...] = jax.lax.dot_general(
        adj_ref[...], y_ref[...],
        dimension_numbers=(((1,), (0,)), ((), ())),
        precision=jax.lax.Precision.DEFAULT,
        preferred_element_type=jnp.float32)


def kernel(input, adj, weight):
    n, d_in = input.shape
    m = adj.shape[0]
    d_out = weight.shape[1]

    return pl.pallas_call(
        _gc_body,
        grid=(m // _BM,),
        in_specs=[
            pl.BlockSpec((_BM, n), lambda i: (i, 0)),
            pl.BlockSpec((n, d_in), lambda i: (0, 0)),
            pl.BlockSpec((d_in, d_out), lambda i: (0, 0)),
        ],
        out_specs=pl.BlockSpec((_BM, d_out), lambda i: (i, 0)),
        out_shape=jax.ShapeDtypeStruct((m, d_out), jnp.float32),
        scratch_shapes=[pltpu.VMEM((n, d_out), jnp.float32)],
        compiler_params=pltpu.CompilerParams(
            dimension_semantics=("arbitrary",)),
    )(adj, input, weight)
